# Initial kernel scaffold; baseline (speedup 1.0000x reference)
import jax
import jax.numpy as jnp
from jax.experimental import pallas as pl


def kernel(x, inj_idx, inj_val, inj_is_add):
    raise NotImplementedError("write your pallas kernel here")


# speedup vs baseline: 1.1070x; 1.1070x over previous
import functools

import jax
import jax.numpy as jnp
from jax import lax
from jax.experimental import pallas as pl
from jax.experimental.pallas import tpu as pltpu
from jax.experimental.pallas import tpu_sc as plsc

N_CORES = 2
N_SUBCORES = 16
N_WORKERS = N_CORES * N_SUBCORES
LANES = 16


@functools.cache
def _make_inject(n_inj: int):
    chunk = n_inj // N_WORKERS
    mesh = plsc.VectorSubcoreMesh(core_axis_name="c", subcore_axis_name="s")

    @functools.partial(
        pl.kernel,
        mesh=mesh,
        out_type=(),
        scratch_types=[
            pltpu.VMEM((chunk,), jnp.int32),
            pltpu.VMEM((chunk,), jnp.float32),
            pltpu.VMEM((chunk,), jnp.float32),
            pltpu.VMEM((chunk,), jnp.float32),
            pltpu.SemaphoreType.DMA,
        ],
    )
    def inject(data_ref, idx_hbm, val_hbm, mask_hbm,
               idx_v, val_v, mask_v, cur_v, sem):
        wid = lax.axis_index("s") * N_CORES + lax.axis_index("c")
        base = wid * chunk
        pltpu.sync_copy(idx_hbm.at[pl.ds(base, chunk)], idx_v)
        pltpu.sync_copy(val_hbm.at[pl.ds(base, chunk)], val_v)
        pltpu.sync_copy(mask_hbm.at[pl.ds(base, chunk)], mask_v)
        pltpu.async_copy(data_ref.at[idx_v], cur_v, sem).wait()
        for i in range(chunk // LANES):
            s = pl.ds(i * LANES, LANES)
            cur_v[s] = val_v[s] + cur_v[s] * mask_v[s]
        pltpu.async_copy(cur_v, data_ref.at[idx_v], sem).wait()

    return inject


def kernel(x, inj_idx, inj_val, inj_is_add):
    idx = inj_idx.astype(jnp.int32)
    mask = inj_is_add.astype(jnp.float32)
    data = jax.new_ref(x.reshape(-1))
    _make_inject(idx.shape[0])(data, idx, inj_val, mask)
    return data[...].reshape(x.shape)


# speedup vs baseline: 7.4227x; 6.7055x over previous
import functools

import jax
import jax.numpy as jnp
from jax import lax
from jax.experimental import pallas as pl
from jax.experimental.pallas import tpu as pltpu
from jax.experimental.pallas import tpu_sc as plsc

N_CORES = 2
N_SUBCORES = 16
N_WORKERS = N_CORES * N_SUBCORES
LANES = 16
TILE_R = 8
TILE_C = 128


@functools.cache
def _make_inject(n_inj: int, n_rows: int, d: int):
    assert n_inj % (N_WORKERS * LANES) == 0
    assert d & (d - 1) == 0
    log2_d = d.bit_length() - 1
    chunk = n_inj // N_WORKERS
    mesh = plsc.VectorSubcoreMesh(core_axis_name="c", subcore_axis_name="s")

    @functools.partial(
        pl.kernel,
        mesh=mesh,
        out_type=(),
        scratch_types=[
            pltpu.VMEM((chunk,), jnp.int32),
            pltpu.VMEM((chunk,), jnp.float32),
            pltpu.VMEM((chunk,), jnp.float32),
            pltpu.VMEM((chunk * TILE_R, TILE_C), jnp.float32),
            pltpu.SemaphoreType.DMA,
        ],
        compiler_params=pltpu.CompilerParams(
            use_tc_tiling_on_sc=True, needs_layout_passes=False),
    )
    def inject(data_ref, idx_hbm, val_hbm, mask_hbm,
               idx_v, val_v, mask_v, tiles_v, sem):
        wid = lax.axis_index("s") * N_CORES + lax.axis_index("c")
        base = wid * chunk
        pltpu.sync_copy(idx_hbm.at[pl.ds(base, chunk)], idx_v)
        pltpu.sync_copy(val_hbm.at[pl.ds(base, chunk)], val_v)
        pltpu.sync_copy(mask_hbm.at[pl.ds(base, chunk)], mask_v)

        def bases_of(j):
            s = pl.ds((j // LANES) * LANES, LANES)
            lane = lax.iota(jnp.int32, LANES) == (j % LANES)
            f = idx_v[s]
            rb16 = jnp.bitwise_and(jnp.right_shift(f, 3), (d // TILE_R) - 1)
            cb16 = jnp.right_shift(f, log2_d + 7)
            rb = lax.reduce_max(jnp.where(lane, rb16, 0), axes=(0,))
            cb = lax.reduce_max(jnp.where(lane, cb16, 0), axes=(0,))
            return rb * TILE_R, cb * TILE_C

        copies = []
        for j in range(chunk):
            rb, cb = bases_of(j)
            copies.append(pltpu.async_copy(
                data_ref.at[pl.ds(rb, TILE_R), pl.ds(cb, TILE_C)],
                tiles_v.at[pl.ds(j * TILE_R, TILE_R), :], sem))
        for c in copies:
            c.wait()

        for i in range(chunk // LANES):
            s = pl.ds(i * LANES, LANES)
            k = lax.iota(jnp.int32, LANES) + i * LANES
            f = idx_v[s]
            row = k * TILE_R + jnp.bitwise_and(f, TILE_R - 1)
            col = jnp.bitwise_and(jnp.right_shift(f, log2_d), TILE_C - 1)
            cur = plsc.load_gather(tiles_v, [row, col])
            plsc.store_scatter(tiles_v, [row, col],
                               val_v[s] + cur * mask_v[s])

        copies = []
        for j in range(chunk):
            rb, cb = bases_of(j)
            copies.append(pltpu.async_copy(
                tiles_v.at[pl.ds(j * TILE_R, TILE_R), :],
                data_ref.at[pl.ds(rb, TILE_R), pl.ds(cb, TILE_C)], sem))
        for c in copies:
            c.wait()

    return inject


def kernel(x, inj_idx, inj_val, inj_is_add):
    idx = inj_idx.astype(jnp.int32)
    mask = inj_is_add.astype(jnp.float32)
    data = jax.new_ref(x.T)
    _make_inject(idx.shape[0], *x.shape)(data, idx, inj_val, mask)
    return data[...].T
